# baseline (speedup 1.0000x reference)
import functools

import jax
import jax.numpy as jnp
from jax import lax
from jax.experimental import pallas as pl
from jax.experimental.pallas import tpu as pltpu
from jax.experimental.pallas import tpu_sc as plsc

_D = 16
_S = 26
_B = 16384
_MAXN = 1.0 - 0.001
_NC, _NS = 2, 16
_NW = _NC * _NS
_B_TOT = _B * _S
_R_PER_W = _B_TOT // _NW
_BW = _B // _NW
_CB = 128
_CHUNK = _CB * _S
_NCHUNK = _BW // _CB
_SUB = 416
_NSUB = _CHUNK // _SUB

_mesh = plsc.VectorSubcoreMesh(
    core_axis_name="c", subcore_axis_name="s",
    num_cores=_NC, num_subcores=_NS)


def _rsqrt(n2):
    i = plsc.bitcast(n2, jnp.int32)
    i = jnp.int32(0x5F3759DF) - (i >> 1)
    y = plsc.bitcast(i, jnp.float32)
    for _ in range(3):
        y = y * (1.5 - 0.5 * n2 * y * y)
    return y


@functools.partial(
    pl.kernel,
    out_type=jax.ShapeDtypeStruct((_S * _D, _B), jnp.float32),
    mesh=_mesh,
    compiler_params=pltpu.CompilerParams(
        needs_layout_passes=False, use_tc_tiling_on_sc=False),
    scratch_types=[
        pltpu.VMEM((_CHUNK,), jnp.int32),
        pltpu.VMEM((_SUB,), jnp.int32),
        pltpu.VMEM((_SUB,), jnp.int32),
        pltpu.VMEM((_SUB, 128), jnp.float32),
        pltpu.VMEM((_S * _D, _CB), jnp.float32),
        pltpu.SemaphoreType.DMA,
    ],
)
def _emb_lookup(x_hbm, w_hbm, out_hbm, idx_v, hi_v, lo_v, rows_v, trans_v,
                sem):
    wid = lax.axis_index("s") * _NC + lax.axis_index("c")
    iota16 = lax.iota(jnp.int32, 16)

    for c in range(_NCHUNK):
        off = wid * _R_PER_W + c * _CHUNK
        b0 = wid * _BW + c * _CB
        pltpu.sync_copy(x_hbm.at[pl.ds(off, _CHUNK)], idx_v)

        for sub in range(_NSUB):
            def split(g, carry, sub=sub):
                sl = pl.ds(g * 16, 16)
                v = idx_v[pl.ds(sub * _SUB + g * 16, 16)]
                hi_v[sl] = v >> 3
                lo_v[sl] = (v & 7) * _D
                return carry

            lax.fori_loop(0, _SUB // 16, split, 0)
            pltpu.async_copy(w_hbm.at[hi_v], rows_v, sem).wait()

            def group(g, carry, sub=sub):
                j = sub * _SUB + g * 16 + iota16
                bl = j // _S
                s16 = (j - bl * _S) * _D
                lo = lo_v[pl.ds(g * 16, 16)]
                jj = g * 16 + iota16
                cols = []
                n2 = jnp.zeros((16,), jnp.float32)
                for d in range(_D):
                    col = plsc.load_gather(rows_v, [jj, lo + d])
                    n2 = n2 + col * col
                    cols.append(col)
                scale = jnp.where(n2 > _MAXN * _MAXN,
                                  _MAXN * _rsqrt(n2), 1.0)
                for d in range(_D):
                    plsc.store_scatter(trans_v, [s16 + d, bl],
                                       cols[d] * scale)
                return carry

            lax.fori_loop(0, _SUB // 16, group, 0)

        pltpu.sync_copy(trans_v, out_hbm.at[:, pl.ds(b0, _CB)])


def kernel(x, W):
    xf = x.reshape(-1).astype(jnp.int32)
    w2 = W.reshape(125000, 128)
    out = _emb_lookup(xf, w2)
    return out.reshape(_S, _D, _B).transpose(2, 0, 1)


# speedup vs baseline: 1.0404x; 1.0404x over previous
import functools

import jax
import jax.numpy as jnp
from jax import lax
from jax.experimental import pallas as pl
from jax.experimental.pallas import tpu as pltpu
from jax.experimental.pallas import tpu_sc as plsc

_D = 16
_S = 26
_B = 16384
_MAXN = 1.0 - 0.001
_NC, _NS = 2, 16
_NW = _NC * _NS
_B_TOT = _B * _S
_R_PER_W = _B_TOT // _NW
_BW = _B // _NW
_CB = 128
_CHUNK = _CB * _S
_NCHUNK = _BW // _CB
_SUB = 416
_NSUB = _CHUNK // _SUB

_mesh = plsc.VectorSubcoreMesh(
    core_axis_name="c", subcore_axis_name="s",
    num_cores=_NC, num_subcores=_NS)


def _rsqrt(n2):
    i = plsc.bitcast(n2, jnp.int32)
    i = jnp.int32(0x5F3759DF) - (i >> 1)
    y = plsc.bitcast(i, jnp.float32)
    for _ in range(3):
        y = y * (1.5 - 0.5 * n2 * y * y)
    return y


@functools.partial(
    pl.kernel,
    out_type=jax.ShapeDtypeStruct((_S * _D, _B), jnp.float32),
    mesh=_mesh,
    compiler_params=pltpu.CompilerParams(
        needs_layout_passes=False, use_tc_tiling_on_sc=True),
    scratch_types=[
        pltpu.VMEM((_CHUNK,), jnp.int32),
        pltpu.VMEM((_SUB,), jnp.int32),
        pltpu.VMEM((_SUB,), jnp.int32),
        pltpu.VMEM((_SUB, 128), jnp.float32),
        pltpu.VMEM((_S * _D, _CB), jnp.float32),
        pltpu.SemaphoreType.DMA,
    ],
)
def _emb_lookup(x_hbm, w_hbm, out_hbm, idx_v, hi_v, lo_v, rows_v, trans_v,
                sem):
    wid = lax.axis_index("s") * _NC + lax.axis_index("c")
    iota16 = lax.iota(jnp.int32, 16)

    for c in range(_NCHUNK):
        off = wid * _R_PER_W + c * _CHUNK
        b0 = wid * _BW + c * _CB
        pltpu.sync_copy(x_hbm.at[pl.ds(off, _CHUNK)], idx_v)

        for sub in range(_NSUB):
            def split(g, carry, sub=sub):
                sl = pl.ds(g * 16, 16)
                v = idx_v[pl.ds(sub * _SUB + g * 16, 16)]
                hi_v[sl] = v >> 3
                lo_v[sl] = (v & 7) * _D
                return carry

            lax.fori_loop(0, _SUB // 16, split, 0)
            pltpu.async_copy(w_hbm.at[hi_v], rows_v, sem).wait()

            def group(g, carry, sub=sub):
                j = sub * _SUB + g * 16 + iota16
                bl = j // _S
                s16 = (j - bl * _S) * _D
                lo = lo_v[pl.ds(g * 16, 16)]
                jj = g * 16 + iota16
                cols = []
                n2 = jnp.zeros((16,), jnp.float32)
                for d in range(_D):
                    col = plsc.load_gather(rows_v, [jj, lo + d])
                    n2 = n2 + col * col
                    cols.append(col)
                scale = jnp.where(n2 > _MAXN * _MAXN,
                                  _MAXN * _rsqrt(n2), 1.0)
                for d in range(_D):
                    plsc.store_scatter(trans_v, [s16 + d, bl],
                                       cols[d] * scale)
                return carry

            lax.fori_loop(0, _SUB // 16, group, 0)

        pltpu.sync_copy(trans_v, out_hbm.at[:, pl.ds(b0, _CB)])


def kernel(x, W):
    xf = x.reshape(-1).astype(jnp.int32)
    w2 = W.reshape(125000, 128)
    out = _emb_lookup(xf, w2)
    return out.reshape(_S, _D, _B).transpose(2, 0, 1)


# speedup vs baseline: 1.1129x; 1.0697x over previous
import functools

import jax
import jax.numpy as jnp
from jax import lax
from jax.experimental import pallas as pl
from jax.experimental.pallas import tpu as pltpu
from jax.experimental.pallas import tpu_sc as plsc

_D = 16
_S = 26
_B = 16384
_N = 1000000
_MAXN = 1.0 - 0.001
_NC, _NS = 2, 16
_NW = _NC * _NS
_B_TOT = _B * _S
_R_PER_W = _B_TOT // _NW
_BW = _B // _NW
_CB = 64
_CHUNK = _CB * _S
_NCHUNK = _BW // _CB

_TBLK = 512
_NFULL = (_N // _TBLK)
_TAIL0 = _NFULL * _TBLK
_BLK_PER_W = -(-_NFULL // _NW)

_mesh = plsc.VectorSubcoreMesh(
    core_axis_name="c", subcore_axis_name="s",
    num_cores=_NC, num_subcores=_NS)


def _rsqrt(n2):
    i = plsc.bitcast(n2, jnp.int32)
    i = jnp.int32(0x5F3759DF) - (i >> 1)
    y = plsc.bitcast(i, jnp.float32)
    for _ in range(3):
        y = y * (1.5 - 0.5 * n2 * y * y)
    return y


@functools.partial(
    pl.kernel,
    out_type=jax.ShapeDtypeStruct((_N * _D // 128, 128), jnp.float32),
    mesh=_mesh,
    compiler_params=pltpu.CompilerParams(
        needs_layout_passes=False, use_tc_tiling_on_sc=True),
    scratch_types=[
        pltpu.VMEM((_D, _TBLK), jnp.float32),
        pltpu.VMEM((_TBLK * _D // 128, 128), jnp.float32),
        pltpu.VMEM((8, 128), jnp.float32),
    ],
)
def _transpose_w(wt_hbm, tail_hbm, ws_hbm, blk_v, out_v, tail_v):
    wid = lax.axis_index("s") * _NC + lax.axis_index("c")
    iota16 = lax.iota(jnp.int32, 16)

    def do_block(i, carry):
        blk = i * _NW + wid

        @pl.when(blk < _NFULL)
        def _():
            c0 = pl.multiple_of(blk * _TBLK, _TBLK)
            pltpu.sync_copy(wt_hbm.at[:, pl.ds(c0, _TBLK)], blk_v)

            def row(r, carry2):
                for g in range(8):
                    v = plsc.load_gather(
                        blk_v, [iota16, jnp.full((16,), 0, jnp.int32)
                                + (8 * r + g)])
                    out_v[r, pl.ds(g * 16, 16)] = v
                return carry2

            lax.fori_loop(0, _TBLK // 8, row, 0)
            r0 = pl.multiple_of(blk * (_TBLK // 8), _TBLK // 8)
            pltpu.sync_copy(out_v, ws_hbm.at[pl.ds(r0, _TBLK // 8), :])

        return carry

    lax.fori_loop(0, _BLK_PER_W, do_block, 0)

    @pl.when(wid == _NW - 1)
    def _():
        pltpu.sync_copy(tail_hbm, tail_v)
        pltpu.sync_copy(tail_v, ws_hbm.at[pl.ds(_TAIL0 * _D // 128, 8), :])


@functools.partial(
    pl.kernel,
    out_type=jax.ShapeDtypeStruct((_S * _D, _B), jnp.float32),
    mesh=_mesh,
    compiler_params=pltpu.CompilerParams(
        needs_layout_passes=False, use_tc_tiling_on_sc=False),
    scratch_types=[
        pltpu.VMEM((2, 1, _CHUNK), jnp.int32),
        pltpu.VMEM((2, _CHUNK, _D), jnp.float32),
        pltpu.VMEM((2, _S * _D, _CB), jnp.float32),
        pltpu.SemaphoreType.DMA,
        pltpu.SemaphoreType.DMA,
        pltpu.SemaphoreType.DMA,
        pltpu.SemaphoreType.DMA,
    ],
)
def _emb_lookup(x_hbm, w_hbm, out_hbm, idx_v, rows_v, trans_v,
                semg0, semg1, semo0, semo1):
    wid = lax.axis_index("s") * _NC + lax.axis_index("c")
    iota16 = lax.iota(jnp.int32, 16)
    semg = (semg0, semg1)
    semo = (semo0, semo1)

    def start_gather(c):
        b = c & 1
        off = wid * _R_PER_W + c * _CHUNK
        pltpu.sync_copy(x_hbm.at[pl.ds(off, _CHUNK)], idx_v.at[b, 0])
        return pltpu.async_copy(w_hbm.at[idx_v.at[b, 0]], rows_v.at[b],
                                semg[b])

    def compute(c):
        b = c & 1

        def group(g, carry):
            j = g * 16 + iota16
            bl = j // _S
            s16 = (j - bl * _S) * _D
            cols = []
            n2 = jnp.zeros((16,), jnp.float32)
            for d in range(_D):
                dsplat = jnp.full((16,), d, jnp.int32)
                col = plsc.load_gather(rows_v.at[b], [j, dsplat])
                n2 = n2 + col * col
                cols.append(col)
            scale = jnp.where(n2 > _MAXN * _MAXN, _MAXN * _rsqrt(n2), 1.0)
            for d in range(_D):
                plsc.store_scatter(trans_v.at[b], [s16 + d, bl],
                                   cols[d] * scale)
            return carry

        lax.fori_loop(0, _CHUNK // 16, group, 0)

    def start_out(c):
        b = c & 1
        b0 = wid * _BW + c * _CB
        return pltpu.async_copy(trans_v.at[b], out_hbm.at[:, pl.ds(b0, _CB)],
                                semo[b])

    hg = [None, None]
    ho = [None, None]
    hg[0] = start_gather(0)
    for c in range(_NCHUNK):
        b = c & 1
        if c + 1 < _NCHUNK:
            hg[1 - b] = start_gather(c + 1)
        hg[b].wait()
        if ho[b] is not None:
            ho[b].wait()
        compute(c)
        ho[b] = start_out(c)
    ho[0].wait()
    ho[1].wait()


def kernel(x, W):
    xf = x.reshape(-1).astype(jnp.int32)
    wt = W.T
    wtail = W[_TAIL0:].reshape(8, 128)
    ws = _transpose_w(wt, wtail)
    w_lin = ws.reshape(_N, _D)
    out = _emb_lookup(xf, w_lin)
    return out.reshape(_S, _D, _B).transpose(2, 0, 1)


# speedup vs baseline: 1.2485x; 1.1218x over previous
import functools

import jax
import jax.numpy as jnp
from jax import lax
from jax.experimental import pallas as pl
from jax.experimental.pallas import tpu as pltpu
from jax.experimental.pallas import tpu_sc as plsc

_D = 16
_S = 26
_B = 16384
_N = 1000000
_MAXN = 1.0 - 0.001
_NC, _NS = 2, 16
_NW = _NC * _NS
_B_TOT = _B * _S
_R_PER_W = _B_TOT // _NW
_BW = _B // _NW
_CB = 64
_CHUNK = _CB * _S
_NCHUNK = _BW // _CB

_TBLK = 1024
_NFULL = (_N // _TBLK)
_TAIL0 = _NFULL * _TBLK
_NSLOT = -(-_NFULL // _NW)
_TAILR = (_N - _TAIL0) * _D // 128

_mesh = plsc.VectorSubcoreMesh(
    core_axis_name="c", subcore_axis_name="s",
    num_cores=_NC, num_subcores=_NS)


def _rsqrt(n2):
    i = plsc.bitcast(n2, jnp.int32)
    i = jnp.int32(0x5F3759DF) - (i >> 1)
    y = plsc.bitcast(i, jnp.float32)
    for _ in range(3):
        y = y * (1.5 - 0.5 * n2 * y * y)
    return y


@functools.partial(
    pl.kernel,
    out_type=jax.ShapeDtypeStruct((_N * _D // 128, 128), jnp.float32),
    mesh=_mesh,
    compiler_params=pltpu.CompilerParams(
        needs_layout_passes=False, use_tc_tiling_on_sc=True),
    scratch_types=[
        pltpu.VMEM((2, _D, _TBLK), jnp.float32),
        pltpu.VMEM((2, _TBLK * _D // 128, 128), jnp.float32),
        pltpu.VMEM((_TAILR, 128), jnp.float32),
        pltpu.SemaphoreType.DMA,
        pltpu.SemaphoreType.DMA,
        pltpu.SemaphoreType.DMA,
        pltpu.SemaphoreType.DMA,
    ],
)
def _transpose_w(wt_hbm, tail_hbm, ws_hbm, blk_v, out_v, tail_v,
                 semi0, semi1, semo0, semo1):
    wid = lax.axis_index("s") * _NC + lax.axis_index("c")
    iota16 = lax.iota(jnp.int32, 16)
    semi = (semi0, semi1)
    semo = (semo0, semo1)

    def _blk(slot):
        return jnp.minimum(slot * _NW + wid, _NFULL - 1)

    def in_copy(slot):
        b = slot & 1
        c0 = pl.multiple_of(_blk(slot) * _TBLK, _TBLK)
        return pltpu.async_copy(wt_hbm.at[:, pl.ds(c0, _TBLK)],
                                blk_v.at[b], semi[b])

    def out_copy(slot):
        b = slot & 1
        r0 = pl.multiple_of(_blk(slot) * (_TBLK // 8), _TBLK // 8)
        return pltpu.async_copy(out_v.at[b],
                                ws_hbm.at[pl.ds(r0, _TBLK // 8), :], semo[b])

    def compute(slot):
        b = slot & 1

        def row(r, carry):
            for g in range(8):
                col = jnp.full((16,), 0, jnp.int32) + (8 * r + g)
                v = plsc.load_gather(blk_v.at[b], [iota16, col])
                out_v[b, r, pl.ds(g * 16, 16)] = v
            return carry

        lax.fori_loop(0, _TBLK // 8, row, 0)

    hin = [None, None]
    hout = [None, None]
    hin[0] = in_copy(0)
    for s in range(_NSLOT):
        b = s & 1
        if s + 1 < _NSLOT:
            hin[1 - b] = in_copy(s + 1)
        hin[b].wait()
        if hout[b] is not None:
            hout[b].wait()
        compute(s)
        hout[b] = out_copy(s)
    hout[0].wait()
    hout[1].wait()

    @pl.when(wid == _NW - 1)
    def _():
        pltpu.sync_copy(tail_hbm, tail_v)
        pltpu.sync_copy(tail_v,
                        ws_hbm.at[pl.ds(_TAIL0 * _D // 128, _TAILR), :])


@functools.partial(
    pl.kernel,
    out_type=jax.ShapeDtypeStruct((_S * _D, _B), jnp.float32),
    mesh=_mesh,
    compiler_params=pltpu.CompilerParams(
        needs_layout_passes=False, use_tc_tiling_on_sc=False),
    scratch_types=[
        pltpu.VMEM((2, 1, _CHUNK), jnp.int32),
        pltpu.VMEM((2, _CHUNK, _D), jnp.float32),
        pltpu.VMEM((2, _S * _D, _CB), jnp.float32),
        pltpu.SemaphoreType.DMA,
        pltpu.SemaphoreType.DMA,
        pltpu.SemaphoreType.DMA,
        pltpu.SemaphoreType.DMA,
    ],
)
def _emb_lookup(x_hbm, w_hbm, out_hbm, idx_v, rows_v, trans_v,
                semg0, semg1, semo0, semo1):
    wid = lax.axis_index("s") * _NC + lax.axis_index("c")
    iota16 = lax.iota(jnp.int32, 16)
    semg = (semg0, semg1)
    semo = (semo0, semo1)

    def start_gather(c):
        b = c & 1
        off = wid * _R_PER_W + c * _CHUNK
        pltpu.sync_copy(x_hbm.at[pl.ds(off, _CHUNK)], idx_v.at[b, 0])
        return pltpu.async_copy(w_hbm.at[idx_v.at[b, 0]], rows_v.at[b],
                                semg[b])

    def compute(c):
        b = c & 1

        def group(g, carry):
            j = g * 16 + iota16
            bl = j // _S
            s16 = (j - bl * _S) * _D
            cols = []
            n2 = jnp.zeros((16,), jnp.float32)
            for d in range(_D):
                dsplat = jnp.full((16,), d, jnp.int32)
                col = plsc.load_gather(rows_v.at[b], [j, dsplat])
                n2 = n2 + col * col
                cols.append(col)
            scale = jnp.where(n2 > _MAXN * _MAXN, _MAXN * _rsqrt(n2), 1.0)
            for d in range(_D):
                plsc.store_scatter(trans_v.at[b], [s16 + d, bl],
                                   cols[d] * scale)
            return carry

        lax.fori_loop(0, _CHUNK // 16, group, 0)

    def start_out(c):
        b = c & 1
        b0 = wid * _BW + c * _CB
        return pltpu.async_copy(trans_v.at[b], out_hbm.at[:, pl.ds(b0, _CB)],
                                semo[b])

    hg = [None, None]
    ho = [None, None]
    hg[0] = start_gather(0)
    for c in range(_NCHUNK):
        b = c & 1
        if c + 1 < _NCHUNK:
            hg[1 - b] = start_gather(c + 1)
        hg[b].wait()
        if ho[b] is not None:
            ho[b].wait()
        compute(c)
        ho[b] = start_out(c)
    ho[0].wait()
    ho[1].wait()


def kernel(x, W):
    xf = x.reshape(-1).astype(jnp.int32)
    wt = W.T
    wtail = W[_TAIL0:].reshape(_TAILR, 128)
    ws = _transpose_w(wt, wtail)
    w_lin = ws.reshape(_N, _D)
    out = _emb_lookup(xf, w_lin)
    return out.reshape(_S, _D, _B).transpose(2, 0, 1)


# speedup vs baseline: 2.2390x; 1.7933x over previous
import functools

import jax
import jax.numpy as jnp
from jax import lax
from jax.experimental import pallas as pl
from jax.experimental.pallas import tpu as pltpu
from jax.experimental.pallas import tpu_sc as plsc

_D = 16
_S = 26
_B = 16384
_N = 1000000
_MAXN = 1.0 - 0.001
_NC, _NS = 2, 16
_NW = _NC * _NS
_B_TOT = _B * _S
_R_PER_W = _B_TOT // _NW
_BW = _B // _NW
_CB = 64
_CHUNK = _CB * _S
_NCHUNK = _BW // _CB

_TBLK = 1024
_NFULL = (_N // _TBLK)
_TAIL0 = _NFULL * _TBLK
_NSLOT = -(-_NFULL // _NW)
_TAILR = (_N - _TAIL0) * _D // 128

_mesh = plsc.VectorSubcoreMesh(
    core_axis_name="c", subcore_axis_name="s",
    num_cores=_NC, num_subcores=_NS)


def _rsqrt(n2):
    i = plsc.bitcast(n2, jnp.int32)
    i = jnp.int32(0x5F3759DF) - (i >> 1)
    y = plsc.bitcast(i, jnp.float32)
    for _ in range(3):
        y = y * (1.5 - 0.5 * n2 * y * y)
    return y


@functools.partial(
    pl.kernel,
    out_type=jax.ShapeDtypeStruct((_N * _D // 128, 128), jnp.float32),
    mesh=_mesh,
    compiler_params=pltpu.CompilerParams(
        needs_layout_passes=False, use_tc_tiling_on_sc=True),
    scratch_types=[
        pltpu.VMEM((2, _D, _TBLK), jnp.float32),
        pltpu.VMEM((2, _TBLK * _D // 128, 128), jnp.float32),
        pltpu.VMEM((_TAILR, 128), jnp.float32),
        pltpu.SemaphoreType.DMA,
        pltpu.SemaphoreType.DMA,
        pltpu.SemaphoreType.DMA,
        pltpu.SemaphoreType.DMA,
    ],
)
def _transpose_w(wt_hbm, tail_hbm, ws_hbm, blk_v, out_v, tail_v,
                 semi0, semi1, semo0, semo1):
    wid = lax.axis_index("s") * _NC + lax.axis_index("c")
    iota16 = lax.iota(jnp.int32, 16)
    semi = (semi0, semi1)
    semo = (semo0, semo1)

    def _blk(slot):
        return jnp.minimum(slot * _NW + wid, _NFULL - 1)

    def in_copy(slot):
        b = slot & 1
        c0 = pl.multiple_of(_blk(slot) * _TBLK, _TBLK)
        return pltpu.async_copy(wt_hbm.at[:, pl.ds(c0, _TBLK)],
                                blk_v.at[b], semi[b])

    def out_copy(slot):
        b = slot & 1
        r0 = pl.multiple_of(_blk(slot) * (_TBLK // 8), _TBLK // 8)
        return pltpu.async_copy(out_v.at[b],
                                ws_hbm.at[pl.ds(r0, _TBLK // 8), :], semo[b])

    def compute(slot):
        b = slot & 1

        def col16(cg, carry):
            cols = cg * 16 + iota16
            rowv = cols >> 3
            cvbase = (cols & 7) * _D
            for d in range(_D):
                v = blk_v[b, d, pl.ds(cg * 16, 16)]
                plsc.store_scatter(out_v.at[b], [rowv, cvbase + d], v)
            return carry

        lax.fori_loop(0, _TBLK // 16, col16, 0)

    hin = [None, None]
    hout = [None, None]
    hin[0] = in_copy(0)
    for s in range(_NSLOT):
        b = s & 1
        if s + 1 < _NSLOT:
            hin[1 - b] = in_copy(s + 1)
        hin[b].wait()
        if hout[b] is not None:
            hout[b].wait()
        compute(s)
        hout[b] = out_copy(s)
    hout[0].wait()
    hout[1].wait()

    @pl.when(wid == _NW - 1)
    def _():
        pltpu.sync_copy(tail_hbm, tail_v)
        pltpu.sync_copy(tail_v,
                        ws_hbm.at[pl.ds(_TAIL0 * _D // 128, _TAILR), :])


@functools.partial(
    pl.kernel,
    out_type=jax.ShapeDtypeStruct((_S * _D, _B), jnp.float32),
    mesh=_mesh,
    compiler_params=pltpu.CompilerParams(
        needs_layout_passes=False, use_tc_tiling_on_sc=False),
    scratch_types=[
        pltpu.VMEM((2, 1, _CHUNK), jnp.int32),
        pltpu.VMEM((2, _CHUNK, _D), jnp.float32),
        pltpu.VMEM((2, _S * _D, _CB), jnp.float32),
        pltpu.SemaphoreType.DMA,
        pltpu.SemaphoreType.DMA,
        pltpu.SemaphoreType.DMA,
        pltpu.SemaphoreType.DMA,
    ],
)
def _emb_lookup(x_hbm, w_hbm, out_hbm, idx_v, rows_v, trans_v,
                semg0, semg1, semo0, semo1):
    wid = lax.axis_index("s") * _NC + lax.axis_index("c")
    iota16 = lax.iota(jnp.int32, 16)
    semg = (semg0, semg1)
    semo = (semo0, semo1)

    def start_gather(c):
        b = c & 1
        off = wid * _R_PER_W + c * _CHUNK
        pltpu.sync_copy(x_hbm.at[pl.ds(off, _CHUNK)], idx_v.at[b, 0])
        return pltpu.async_copy(w_hbm.at[idx_v.at[b, 0]], rows_v.at[b],
                                semg[b])

    def compute(c):
        b = c & 1

        def group(g, carry):
            j = g * 16 + iota16
            bl = j // _S
            s16 = (j - bl * _S) * _D
            cols = []
            n2 = jnp.zeros((16,), jnp.float32)
            for d in range(_D):
                dsplat = jnp.full((16,), d, jnp.int32)
                col = plsc.load_gather(rows_v.at[b], [j, dsplat])
                n2 = n2 + col * col
                cols.append(col)
            scale = jnp.where(n2 > _MAXN * _MAXN, _MAXN * _rsqrt(n2), 1.0)
            for d in range(_D):
                plsc.store_scatter(trans_v.at[b], [s16 + d, bl],
                                   cols[d] * scale)
            return carry

        lax.fori_loop(0, _CHUNK // 16, group, 0)

    def start_out(c):
        b = c & 1
        b0 = wid * _BW + c * _CB
        return pltpu.async_copy(trans_v.at[b], out_hbm.at[:, pl.ds(b0, _CB)],
                                semo[b])

    hg = [None, None]
    ho = [None, None]
    hg[0] = start_gather(0)
    for c in range(_NCHUNK):
        b = c & 1
        if c + 1 < _NCHUNK:
            hg[1 - b] = start_gather(c + 1)
        hg[b].wait()
        if ho[b] is not None:
            ho[b].wait()
        compute(c)
        ho[b] = start_out(c)
    ho[0].wait()
    ho[1].wait()


def kernel(x, W):
    xf = x.reshape(-1).astype(jnp.int32)
    wt = W.T
    wtail = W[_TAIL0:].reshape(_TAILR, 128)
    ws = _transpose_w(wt, wtail)
    w_lin = ws.reshape(_N, _D)
    out = _emb_lookup(xf, w_lin)
    return out.reshape(_S, _D, _B).transpose(2, 0, 1)
